# Initial kernel scaffold; baseline (speedup 1.0000x reference)
import functools

import jax
import jax.numpy as jnp
from jax import lax
from jax.experimental import pallas as pl
from jax.experimental.pallas import tpu as pltpu
from jax.experimental.pallas import tpu_sc as plsc

B = 4096
F = 26
V = 100000
E = 16
VID = 128
L = 50
D1, D2 = 32, 32

NC, NS = 2, 16
NW = NC * NS
SPW = B // NW
CH = 64
G2 = CH * F // 128
GT = CH * L // 128
N2C = CH * F
NTC = CH * L

_MESH = plsc.VectorSubcoreMesh(
    core_axis_name="c", subcore_axis_name="s", num_cores=NC, num_subcores=NS)


@functools.partial(
    pl.kernel,
    out_type=(
        jax.ShapeDtypeStruct((B * F, E), jnp.float32),
        jax.ShapeDtypeStruct((B * L, E), jnp.float32),
        jax.ShapeDtypeStruct((B * F,), jnp.float32),
    ),
    mesh=_MESH,
    scratch_types=(
        pltpu.VMEM((G2, 128), jnp.int32),
        pltpu.VMEM((GT, 128), jnp.int32),
        pltpu.VMEM((N2C, E), jnp.float32),
        pltpu.VMEM((NTC, E), jnp.float32),
        pltpu.VMEM((N2C,), jnp.float32),
        pltpu.SemaphoreType.DMA,
    ),
)
def _sc_gather(idx2_hbm, tidx_hbm, t2_hbm, t1_hbm, tt_hbm,
               out2_hbm, outt_hbm, out1_hbm,
               idx2_v, tidx_v, rows2_v, trows_v, vals1_v, sem):
    wid = lax.axis_index("s") * NC + lax.axis_index("c")
    for ch in range(SPW // CH):
        s0 = wid * SPW + ch * CH
        pltpu.sync_copy(idx2_hbm.at[pl.ds(s0 * F // 128, G2)], idx2_v)
        pltpu.sync_copy(tidx_hbm.at[pl.ds(s0 * L // 128, GT)], tidx_v)
        copies = []
        for g in range(G2):
            copies.append(pltpu.async_copy(
                t2_hbm.at[idx2_v.at[g]], rows2_v.at[pl.ds(g * 128, 128)], sem))
        for g in range(GT):
            copies.append(pltpu.async_copy(
                tt_hbm.at[tidx_v.at[g]], trows_v.at[pl.ds(g * 128, 128)], sem))
        for g in range(G2):
            copies.append(pltpu.async_copy(
                t1_hbm.at[idx2_v.at[g]], vals1_v.at[pl.ds(g * 128, 128)], sem))
        for c in copies:
            c.wait()
        pltpu.sync_copy(rows2_v, out2_hbm.at[pl.ds(s0 * F, N2C)])
        pltpu.sync_copy(trows_v, outt_hbm.at[pl.ds(s0 * L, NTC)])
        pltpu.sync_copy(vals1_v, out1_hbm.at[pl.ds(s0 * F, N2C)])


BB = 512


def _tc_body(sec_ref, title_ref, first_ref, xv_ref, tv_ref, vid_ref,
             vWT_ref, vb_ref, W1aT_ref, W1bT_ref, W1cT_ref, b1_ref,
             W2T_ref, b2_ref, R26_ref, S26_ref, R50_ref, S50_ref, bias_ref,
             out_ref):
    hi = lax.Precision.HIGHEST
    f32 = jnp.float32
    xv = xv_ref[...]
    fm1 = jnp.sum(first_ref[...] * xv, axis=1)
    xvE = jnp.dot(xv, R26_ref[...], precision=hi, preferred_element_type=f32)
    sec = sec_ref[...] * xvE
    tvE = jnp.dot(tv_ref[...], R50_ref[...], precision=hi,
                  preferred_element_type=f32)
    temb = jnp.dot(title_ref[...] * tvE, S50_ref[...], precision=hi,
                   preferred_element_type=f32)
    vemb = jnp.dot(vid_ref[...], vWT_ref[...], precision=hi,
                   preferred_element_type=f32) + vb_ref[...]
    se = jnp.dot(sec, S26_ref[...], precision=hi,
                 preferred_element_type=f32) + temb + vemb
    sq = jnp.dot(sec * sec, S26_ref[...], precision=hi,
                 preferred_element_type=f32) + temb * temb + vemb * vemb
    fm2 = 0.5 * jnp.sum(se * se - sq, axis=1)
    x = jnp.dot(sec, W1aT_ref[...], precision=hi, preferred_element_type=f32)
    x = x + jnp.dot(temb, W1bT_ref[...], precision=hi,
                    preferred_element_type=f32)
    x = x + jnp.dot(vemb, W1cT_ref[...], precision=hi,
                    preferred_element_type=f32)
    x = jnp.maximum(x + b1_ref[...], 0.0)
    x = jnp.maximum(jnp.dot(x, W2T_ref[...], precision=hi,
                            preferred_element_type=f32) + b2_ref[...], 0.0)
    out_ref[...] = fm1 + fm2 + jnp.sum(x, axis=1) + bias_ref[0, 0]


def _row_spec(w):
    return pl.BlockSpec((BB, w), lambda i: (i, 0))


def _full_spec(h, w):
    return pl.BlockSpec((h, w), lambda i: (0, 0))


def kernel(Xi, Xv, video_feature, title_feature, title_value,
           fm_first_tables, fm_second_tables, title_table,
           video_W, video_b, W1, b1, W2, b2, bias):
    f32 = jnp.float32
    Xi32 = Xi[:, :, 0].astype(jnp.int32)
    idx2 = (Xi32 + jnp.arange(F, dtype=jnp.int32)[None, :] * V)
    idx2 = idx2.reshape(B * F // 128, 128)
    tidx = title_feature.astype(jnp.int32).reshape(B * L // 128, 128)
    t2 = fm_second_tables.reshape(F * V, E)
    t1 = fm_first_tables.reshape(F * V)

    sec_raw, title_raw, first_raw = _sc_gather(idx2, tidx, t2, t1, title_table)
    sec_raw = sec_raw.reshape(B, F * E)
    title_raw = title_raw.reshape(B, L * E)
    first_raw = first_raw.reshape(B, F)

    eyeE = jnp.eye(E, dtype=f32)
    R26 = jnp.repeat(jnp.eye(F, dtype=f32), E, axis=1)
    S26 = jnp.tile(eyeE, (F, 1))
    R50 = jnp.repeat(jnp.eye(L, dtype=f32), E, axis=1)
    S50 = jnp.tile(eyeE, (L, 1))
    W1aT = W1[:, :F * E].T
    W1bT = W1[:, F * E:(F + 1) * E].T
    W1cT = W1[:, (F + 1) * E:].T

    out = pl.pallas_call(
        _tc_body,
        grid=(B // BB,),
        in_specs=[
            _row_spec(F * E), _row_spec(L * E), _row_spec(F), _row_spec(F),
            _row_spec(L), _row_spec(VID),
            _full_spec(VID, E), _full_spec(1, E),
            _full_spec(F * E, D1), _full_spec(E, D1), _full_spec(E, D1),
            _full_spec(1, D1), _full_spec(D1, D2), _full_spec(1, D2),
            _full_spec(F, F * E), _full_spec(F * E, E),
            _full_spec(L, L * E), _full_spec(L * E, E),
            _full_spec(1, 1),
        ],
        out_specs=pl.BlockSpec((BB,), lambda i: (i,)),
        out_shape=jax.ShapeDtypeStruct((B,), f32),
    )(sec_raw, title_raw, first_raw, Xv, title_value, video_feature,
      video_W.T, video_b[None, :], W1aT, W1bT, W1cT, b1[None, :],
      W2.T, b2[None, :], R26, S26, R50, S50, bias[None, :])
    return out


# speedup vs baseline: 1.5129x; 1.5129x over previous
import functools

import jax
import jax.numpy as jnp
from jax import lax
from jax.experimental import pallas as pl
from jax.experimental.pallas import tpu as pltpu
from jax.experimental.pallas import tpu_sc as plsc

B = 4096
F = 26
V = 100000
E = 16
VID = 128
L = 50
D1, D2 = 32, 32

NC, NS = 2, 16
NW = NC * NS
SPW = B // NW
CH = 64
G2 = CH * F // 128
GT = CH * L // 128
N2C = CH * F
NTC = CH * L

@functools.lru_cache(maxsize=None)
def _make_sc_gather():
    mesh = plsc.VectorSubcoreMesh(
        core_axis_name="c", subcore_axis_name="s",
        num_cores=NC, num_subcores=NS)

    @functools.partial(
        pl.kernel,
        out_type=(
            jax.ShapeDtypeStruct((B * F, E), jnp.float32),
            jax.ShapeDtypeStruct((B * L, E), jnp.float32),
            jax.ShapeDtypeStruct((B * F,), jnp.float32),
        ),
        mesh=mesh,
        compiler_params=pltpu.CompilerParams(use_tc_tiling_on_sc=False),
        scratch_types=(
            pltpu.VMEM((N2C,), jnp.int32),
            pltpu.VMEM((NTC,), jnp.int32),
            pltpu.VMEM((N2C, E), jnp.float32),
            pltpu.VMEM((NTC, E), jnp.float32),
            pltpu.VMEM((N2C,), jnp.float32),
            pltpu.SemaphoreType.DMA,
        ),
    )
    def _sc_gather(idx2_hbm, tidx_hbm, t2_hbm, t1_hbm, tt_hbm,
                   out2_hbm, outt_hbm, out1_hbm,
                   idx2_v, tidx_v, rows2_v, trows_v, vals1_v, sem):
        wid = lax.axis_index("s") * NC + lax.axis_index("c")
        for ch in range(SPW // CH):
            s0 = wid * SPW + ch * CH
            pltpu.sync_copy(idx2_hbm.at[pl.ds(s0 * F, N2C)], idx2_v)
            pltpu.sync_copy(tidx_hbm.at[pl.ds(s0 * L, NTC)], tidx_v)
            copies = []
            for g in range(G2):
                copies.append(pltpu.async_copy(
                    t2_hbm.at[idx2_v.at[pl.ds(g * 128, 128)]],
                    rows2_v.at[pl.ds(g * 128, 128)], sem))
            for g in range(GT):
                copies.append(pltpu.async_copy(
                    tt_hbm.at[tidx_v.at[pl.ds(g * 128, 128)]],
                    trows_v.at[pl.ds(g * 128, 128)], sem))
            for g in range(G2):
                copies.append(pltpu.async_copy(
                    t1_hbm.at[idx2_v.at[pl.ds(g * 128, 128)]],
                    vals1_v.at[pl.ds(g * 128, 128)], sem))
            for c in copies:
                c.wait()
            pltpu.sync_copy(rows2_v, out2_hbm.at[pl.ds(s0 * F, N2C)])
            pltpu.sync_copy(trows_v, outt_hbm.at[pl.ds(s0 * L, NTC)])
            pltpu.sync_copy(vals1_v, out1_hbm.at[pl.ds(s0 * F, N2C)])

    return _sc_gather


BB = 512


def _tc_body(sec_ref, title_ref, first_ref, xv_ref, tv_ref, vid_ref,
             vWT_ref, vb_ref, W1aT_ref, W1bT_ref, W1cT_ref, b1_ref,
             W2T_ref, b2_ref, R26_ref, S26_ref, R50_ref, S50_ref, bias_ref,
             out_ref):
    hi = lax.Precision.HIGHEST
    f32 = jnp.float32
    xv = xv_ref[...]
    fm1 = jnp.sum(first_ref[...] * xv, axis=1)
    xvE = jnp.dot(xv, R26_ref[...], precision=hi, preferred_element_type=f32)
    sec = sec_ref[...] * xvE
    tvE = jnp.dot(tv_ref[...], R50_ref[...], precision=hi,
                  preferred_element_type=f32)
    temb = jnp.dot(title_ref[...] * tvE, S50_ref[...], precision=hi,
                   preferred_element_type=f32)
    vemb = jnp.dot(vid_ref[...], vWT_ref[...], precision=hi,
                   preferred_element_type=f32) + vb_ref[...]
    se = jnp.dot(sec, S26_ref[...], precision=hi,
                 preferred_element_type=f32) + temb + vemb
    sq = jnp.dot(sec * sec, S26_ref[...], precision=hi,
                 preferred_element_type=f32) + temb * temb + vemb * vemb
    fm2 = 0.5 * jnp.sum(se * se - sq, axis=1)
    x = jnp.dot(sec, W1aT_ref[...], precision=hi, preferred_element_type=f32)
    x = x + jnp.dot(temb, W1bT_ref[...], precision=hi,
                    preferred_element_type=f32)
    x = x + jnp.dot(vemb, W1cT_ref[...], precision=hi,
                    preferred_element_type=f32)
    x = jnp.maximum(x + b1_ref[...], 0.0)
    x = jnp.maximum(jnp.dot(x, W2T_ref[...], precision=hi,
                            preferred_element_type=f32) + b2_ref[...], 0.0)
    out_ref[...] = fm1 + fm2 + jnp.sum(x, axis=1) + bias_ref[0, 0]


def _row_spec(w):
    return pl.BlockSpec((BB, w), lambda i: (i, 0))


def _full_spec(h, w):
    return pl.BlockSpec((h, w), lambda i: (0, 0))


def kernel(Xi, Xv, video_feature, title_feature, title_value,
           fm_first_tables, fm_second_tables, title_table,
           video_W, video_b, W1, b1, W2, b2, bias):
    f32 = jnp.float32
    Xi32 = Xi[:, :, 0].astype(jnp.int32)
    idx2 = (Xi32 + jnp.arange(F, dtype=jnp.int32)[None, :] * V).reshape(B * F)
    tidx = title_feature.astype(jnp.int32).reshape(B * L)
    t2 = fm_second_tables.reshape(F * V, E)
    t1 = fm_first_tables.reshape(F * V)

    sec_raw, title_raw, first_raw = _make_sc_gather()(
        idx2, tidx, t2, t1, title_table)
    sec_raw = sec_raw.reshape(B, F * E)
    title_raw = title_raw.reshape(B, L * E)
    first_raw = first_raw.reshape(B, F)

    eyeE = jnp.eye(E, dtype=f32)
    R26 = jnp.repeat(jnp.eye(F, dtype=f32), E, axis=1)
    S26 = jnp.tile(eyeE, (F, 1))
    R50 = jnp.repeat(jnp.eye(L, dtype=f32), E, axis=1)
    S50 = jnp.tile(eyeE, (L, 1))
    W1aT = W1[:, :F * E].T
    W1bT = W1[:, F * E:(F + 1) * E].T
    W1cT = W1[:, (F + 1) * E:].T

    out = pl.pallas_call(
        _tc_body,
        grid=(B // BB,),
        in_specs=[
            _row_spec(F * E), _row_spec(L * E), _row_spec(F), _row_spec(F),
            _row_spec(L), _row_spec(VID),
            _full_spec(VID, E), _full_spec(1, E),
            _full_spec(F * E, D1), _full_spec(E, D1), _full_spec(E, D1),
            _full_spec(1, D1), _full_spec(D1, D2), _full_spec(1, D2),
            _full_spec(F, F * E), _full_spec(F * E, E),
            _full_spec(L, L * E), _full_spec(L * E, E),
            _full_spec(1, 1),
        ],
        out_specs=pl.BlockSpec((BB,), lambda i: (i,)),
        out_shape=jax.ShapeDtypeStruct((B,), f32),
    )(sec_raw, title_raw, first_raw, Xv, title_value, video_feature,
      video_W.T, video_b[None, :], W1aT, W1bT, W1cT, b1[None, :],
      W2.T, b2[None, :], R26, S26, R50, S50, bias[None, :])
    return out


# speedup vs baseline: 1.6428x; 1.0859x over previous
import functools

import jax
import jax.numpy as jnp
from jax import lax
from jax.experimental import pallas as pl
from jax.experimental.pallas import tpu as pltpu
from jax.experimental.pallas import tpu_sc as plsc

B = 4096
F = 26
V = 100000
E = 16
VID = 128
L = 50
D1, D2 = 32, 32

RP = V * E // 128
RPP = RP + 4
S2 = RPP * 128 // E
N2ROWS = F * S2
T1R = F * V // E

NC, NS = 2, 16
NW = NC * NS
SPW = B // NW
CH = 64
G2 = CH * F // 128
GT = CH * L // 128
N2C = CH * F
NTC = CH * L


@functools.lru_cache(maxsize=None)
def _make_sc_gather():
    mesh = plsc.VectorSubcoreMesh(
        core_axis_name="c", subcore_axis_name="s",
        num_cores=NC, num_subcores=NS)

    @functools.partial(
        pl.kernel,
        out_type=(
            jax.ShapeDtypeStruct((B * F, E), jnp.float32),
            jax.ShapeDtypeStruct((B * L, E), jnp.float32),
            jax.ShapeDtypeStruct((B * F, E), jnp.float32),
        ),
        mesh=mesh,
        compiler_params=pltpu.CompilerParams(use_tc_tiling_on_sc=False),
        scratch_types=(
            pltpu.VMEM((N2C,), jnp.int32),
            pltpu.VMEM((N2C,), jnp.int32),
            pltpu.VMEM((NTC,), jnp.int32),
            pltpu.VMEM((N2C, E), jnp.float32),
            pltpu.VMEM((NTC, E), jnp.float32),
            pltpu.VMEM((N2C, E), jnp.float32),
            pltpu.SemaphoreType.DMA,
        ),
    )
    def _sc_gather(idx2_hbm, idx1_hbm, tidx_hbm, t2_hbm, t1_hbm, tt_hbm,
                   out2_hbm, outt_hbm, out1_hbm,
                   idx2_v, idx1_v, tidx_v, rows2_v, trows_v, rows1_v, sem):
        wid = lax.axis_index("s") * NC + lax.axis_index("c")
        for ch in range(SPW // CH):
            s0 = wid * SPW + ch * CH
            pltpu.sync_copy(idx2_hbm.at[pl.ds(s0 * F, N2C)], idx2_v)
            pltpu.sync_copy(idx1_hbm.at[pl.ds(s0 * F, N2C)], idx1_v)
            pltpu.sync_copy(tidx_hbm.at[pl.ds(s0 * L, NTC)], tidx_v)
            copies = []
            for g in range(G2):
                copies.append(pltpu.async_copy(
                    t2_hbm.at[idx2_v.at[pl.ds(g * 128, 128)]],
                    rows2_v.at[pl.ds(g * 128, 128)], sem))
            for g in range(GT):
                copies.append(pltpu.async_copy(
                    tt_hbm.at[tidx_v.at[pl.ds(g * 128, 128)]],
                    trows_v.at[pl.ds(g * 128, 128)], sem))
            for g in range(G2):
                copies.append(pltpu.async_copy(
                    t1_hbm.at[idx1_v.at[pl.ds(g * 128, 128)]],
                    rows1_v.at[pl.ds(g * 128, 128)], sem))
            for c in copies:
                c.wait()
            pltpu.sync_copy(rows2_v, out2_hbm.at[pl.ds(s0 * F, N2C)])
            pltpu.sync_copy(trows_v, outt_hbm.at[pl.ds(s0 * L, NTC)])
            pltpu.sync_copy(rows1_v, out1_hbm.at[pl.ds(s0 * F, N2C)])

    return _sc_gather


_CHUNKS = [(c * 512, 512) for c in range(RP // 512)] + [(RP - RP % 512,
                                                         RP % 512)]


def _pack_store(load_cols, store_rows):
    for r0, nr in _CHUNKS:
        parts = [jnp.swapaxes(load_cols(RP * t + r0, nr), 0, 1)
                 for t in range(8)]
        store_rows(r0, nr, jnp.concatenate(parts, axis=1))


def _relayout2_body(in_ref, out_ref):
    _pack_store(lambda c0, nc: in_ref[0, :, pl.ds(c0, nc)],
                lambda r0, nr, v: out_ref.__setitem__(
                    (0, pl.ds(r0, nr), slice(None)), v))
    out_ref[0, RP:, :] = jnp.zeros((RPP - RP, 128), jnp.float32)


def _relayout_t_body(in_ref, out_ref):
    _pack_store(lambda c0, nc: in_ref[:, pl.ds(c0, nc)],
                lambda r0, nr, v: out_ref.__setitem__(
                    (pl.ds(r0, nr), slice(None)), v))


def _relayout_tables(t2T, ttT):
    t2p = pl.pallas_call(
        _relayout2_body,
        grid=(F,),
        in_specs=[pl.BlockSpec((1, E, V), lambda i: (i, 0, 0))],
        out_specs=pl.BlockSpec((1, RPP, 128), lambda i: (i, 0, 0)),
        out_shape=jax.ShapeDtypeStruct((F, RPP, 128), jnp.float32),
    )(t2T)
    ttp = pl.pallas_call(
        _relayout_t_body,
        grid=(1,),
        in_specs=[pl.BlockSpec((E, V), lambda j: (0, 0))],
        out_specs=pl.BlockSpec((RP, 128), lambda j: (0, 0)),
        out_shape=jax.ShapeDtypeStruct((RP, 128), jnp.float32),
    )(ttT)
    return t2p, ttp


BB = 512


def _tc_body(sec_ref, title_ref, first_ref, xim_ref, xv_ref, tv_ref, vid_ref,
             vWT_ref, vb_ref, W1aT_ref, W1bT_ref, W1cT_ref, b1_ref,
             W2T_ref, b2_ref, R26_ref, S26_ref, R50_ref, S50_ref, iota_ref,
             bias_ref, out_ref):
    hi = lax.Precision.HIGHEST
    f32 = jnp.float32
    xv = xv_ref[...]
    xvE = jnp.dot(xv, R26_ref[...], precision=hi, preferred_element_type=f32)
    ximE = jnp.dot(xim_ref[...], R26_ref[...], precision=hi,
                   preferred_element_type=f32)
    sel = jnp.where(ximE == iota_ref[...], first_ref[...], 0.0)
    fm1 = jnp.sum(sel * xvE, axis=1)
    sec = sec_ref[...] * xvE
    tvE = jnp.dot(tv_ref[...], R50_ref[...], precision=hi,
                  preferred_element_type=f32)
    temb = jnp.dot(title_ref[...] * tvE, S50_ref[...], precision=hi,
                   preferred_element_type=f32)
    vemb = jnp.dot(vid_ref[...], vWT_ref[...], precision=hi,
                   preferred_element_type=f32) + vb_ref[...]
    se = jnp.dot(sec, S26_ref[...], precision=hi,
                 preferred_element_type=f32) + temb + vemb
    sq = jnp.dot(sec * sec, S26_ref[...], precision=hi,
                 preferred_element_type=f32) + temb * temb + vemb * vemb
    fm2 = 0.5 * jnp.sum(se * se - sq, axis=1)
    x = jnp.dot(sec, W1aT_ref[...], precision=hi, preferred_element_type=f32)
    x = x + jnp.dot(temb, W1bT_ref[...], precision=hi,
                    preferred_element_type=f32)
    x = x + jnp.dot(vemb, W1cT_ref[...], precision=hi,
                    preferred_element_type=f32)
    x = jnp.maximum(x + b1_ref[...], 0.0)
    x = jnp.maximum(jnp.dot(x, W2T_ref[...], precision=hi,
                            preferred_element_type=f32) + b2_ref[...], 0.0)
    out_ref[...] = fm1 + fm2 + jnp.sum(x, axis=1) + bias_ref[0, 0]


def _row_spec(w):
    return pl.BlockSpec((BB, w), lambda i: (i, 0))


def _full_spec(h, w):
    return pl.BlockSpec((h, w), lambda i: (0, 0))


def kernel(Xi, Xv, video_feature, title_feature, title_value,
           fm_first_tables, fm_second_tables, title_table,
           video_W, video_b, W1, b1, W2, b2, bias):
    f32 = jnp.float32
    Xi32 = Xi[:, :, 0].astype(jnp.int32)
    pack16 = (Xi32 % RP) * 8 + Xi32 // RP
    offs2 = jnp.arange(F, dtype=jnp.int32)[None, :] * S2
    idx2 = (pack16 + offs2).reshape(B * F)
    offs1 = jnp.arange(F, dtype=jnp.int32)[None, :] * (V // E)
    idx1 = ((Xi32 >> 4) + offs1).reshape(B * F)
    xim = (Xi32 & 15).astype(f32)
    t32 = title_feature.astype(jnp.int32)
    tidx = ((t32 % RP) * 8 + t32 // RP).reshape(B * L)
    t2T = jnp.transpose(fm_second_tables, (0, 2, 1))
    ttT = jnp.transpose(title_table, (1, 0))
    t2p, ttp = _relayout_tables(t2T, ttT)
    t2 = t2p.reshape(N2ROWS, E)
    tt = ttp.reshape(V, E)
    t1 = fm_first_tables.reshape(T1R, E)

    sec_raw, title_raw, first_raw = _make_sc_gather()(
        idx2, idx1, tidx, t2, t1, tt)
    sec_raw = sec_raw.reshape(B, F * E)
    title_raw = title_raw.reshape(B, L * E)
    first_raw = first_raw.reshape(B, F * E)

    eyeE = jnp.eye(E, dtype=f32)
    R26 = jnp.repeat(jnp.eye(F, dtype=f32), E, axis=1)
    S26 = jnp.tile(eyeE, (F, 1))
    R50 = jnp.repeat(jnp.eye(L, dtype=f32), E, axis=1)
    S50 = jnp.tile(eyeE, (L, 1))
    iota416 = jnp.tile(jnp.arange(E, dtype=f32), F)[None, :]
    W1aT = W1[:, :F * E].T
    W1bT = W1[:, F * E:(F + 1) * E].T
    W1cT = W1[:, (F + 1) * E:].T

    out = pl.pallas_call(
        _tc_body,
        grid=(B // BB,),
        in_specs=[
            _row_spec(F * E), _row_spec(L * E), _row_spec(F * E),
            _row_spec(F), _row_spec(F), _row_spec(L), _row_spec(VID),
            _full_spec(VID, E), _full_spec(1, E),
            _full_spec(F * E, D1), _full_spec(E, D1), _full_spec(E, D1),
            _full_spec(1, D1), _full_spec(D1, D2), _full_spec(1, D2),
            _full_spec(F, F * E), _full_spec(F * E, E),
            _full_spec(L, L * E), _full_spec(L * E, E),
            _full_spec(1, F * E), _full_spec(1, 1),
        ],
        out_specs=pl.BlockSpec((BB,), lambda i: (i,)),
        out_shape=jax.ShapeDtypeStruct((B,), f32),
    )(sec_raw, title_raw, first_raw, xim, Xv, title_value, video_feature,
      video_W.T, video_b[None, :], W1aT, W1bT, W1cT, b1[None, :],
      W2.T, b2[None, :], R26, S26, R50, S50, iota416, bias[None, :])
    return out
